# Initial kernel scaffold; baseline (speedup 1.0000x reference)
import jax
import jax.numpy as jnp
from jax import lax
from jax.experimental import pallas as pl
from jax.experimental.pallas import tpu as pltpu
from jax.experimental.pallas import tpu_sc as plsc

_K = 128
_Y = 16
_C = 128
_N = 320000
_NSEG = _K * _Y
_NC = 2
_NS = 16
_NW = _NC * _NS
_G = 128
_NGROUPS = _N // _G
_BASE_GROUPS = _NGROUPS // _NW
_EXTRA = _NGROUPS - _BASE_GROUPS * _NW
_CG = 3
_CHUNK = _CG * _G
_NCHUNKS = _BASE_GROUPS // _CG
_SEG_PER_TILE = _NSEG // _NS

assert _BASE_GROUPS % _CG == 0
assert _NGROUPS * _G == _N


def _sc_body(x_hbm, y_hbm, p_hbm, z_hbm, out_hbm, xv, yv, idxv, rows, acc):
    cid = lax.axis_index("c")
    sid = lax.axis_index("s")
    wid = sid * _NC + cid

    pltpu.sync_copy(
        z_hbm.at[pl.ds(sid * _SEG_PER_TILE, _SEG_PER_TILE), :],
        acc.at[pl.ds(sid * _SEG_PER_TILE, _SEG_PER_TILE), :],
    )
    plsc.subcore_barrier()

    base_w = pl.multiple_of(
        _G * (_BASE_GROUPS * wid + jnp.minimum(wid, _EXTRA)), _G
    )

    def process(base, ngroups, nrows):
        pltpu.sync_copy(x_hbm.at[pl.ds(base, nrows)], xv.at[pl.ds(0, nrows)])
        pltpu.sync_copy(y_hbm.at[pl.ds(base, nrows)], yv.at[pl.ds(0, nrows)])
        pltpu.sync_copy(p_hbm.at[pl.ds(base, nrows), :],
                        rows.at[pl.ds(0, nrows), :])
        for g in range(ngroups):
            for i in range(_G // 16):
                off = g * _G + i * 16
                seg = xv[pl.ds(off, 16)] * _Y + yv[pl.ds(off, 16)]
                idxv[g, pl.ds(i * 16, 16)] = seg
        for g in range(ngroups):
            pltpu.sync_copy(rows.at[pl.ds(g * _G, _G), :],
                            acc.at[idxv.at[g]], add=True)

    def chunk_body(j, carry):
        process(base_w + j * _CHUNK, _CG, _CHUNK)
        return carry

    lax.fori_loop(0, _NCHUNKS, chunk_body, 0)

    @pl.when(wid < _EXTRA)
    def _tail():
        process(pl.multiple_of(_G * (_BASE_GROUPS * _NW + wid), _G), 1, _G)

    plsc.subcore_barrier()
    pltpu.sync_copy(
        acc.at[pl.ds(sid * _SEG_PER_TILE, _SEG_PER_TILE), :],
        out_hbm.at[cid, pl.ds(sid * _SEG_PER_TILE, _SEG_PER_TILE), :],
    )


_sc_call = pl.kernel(
    _sc_body,
    out_type=jax.ShapeDtypeStruct((_NC, _NSEG, _C), jnp.float32),
    mesh=plsc.VectorSubcoreMesh(core_axis_name="c", subcore_axis_name="s"),
    scratch_types=[
        pltpu.VMEM((_CHUNK,), jnp.int32),
        pltpu.VMEM((_CHUNK,), jnp.int32),
        pltpu.VMEM((_CG, _G), jnp.int32),
        pltpu.VMEM((_CHUNK, _C), jnp.float32),
        pltpu.VMEM_SHARED((_NSEG, _C), jnp.float32),
    ],
)


def _merge_body(em_ref, p_ref, o_ref):
    o_ref[...] = em_ref[...] + p_ref[0] + p_ref[1]


_merge = pl.pallas_call(
    _merge_body,
    out_shape=jax.ShapeDtypeStruct((_NSEG, _C), jnp.float32),
)


def kernel(x_labels, y_labels, posterior_estimate, emission_numerator):
    zeros = jnp.zeros((_NSEG, _C), jnp.float32)
    partials = _sc_call(x_labels, y_labels, posterior_estimate, zeros)
    out = _merge(emission_numerator.reshape(_NSEG, _C), partials)
    return out.reshape(_K, _Y, _C)


# speedup vs baseline: 5.7500x; 5.7500x over previous
import jax
import jax.numpy as jnp
from jax import lax
from jax.experimental import pallas as pl
from jax.experimental.pallas import tpu as pltpu
from jax.experimental.pallas import tpu_sc as plsc

_K = 128
_Y = 16
_C = 128
_N = 320000
_NSEG = _K * _Y
_NC = 2
_NS = 16
_NW = _NC * _NS
_G = 128
_NGROUPS = _N // _G
_BASE_GROUPS = _NGROUPS // _NW
_EXTRA = _NGROUPS - _BASE_GROUPS * _NW
_CG = 3
_CHUNK = _CG * _G
_NCHUNKS = _BASE_GROUPS // _CG
_SEG_PER_TILE = _NSEG // _NS

assert _BASE_GROUPS % _CG == 0
assert _NGROUPS * _G == _N


def _sc_body(x_hbm, y_hbm, p_hbm, z_hbm, out_hbm, xv, yv, idxv, rows, acc):
    cid = lax.axis_index("c")
    sid = lax.axis_index("s")
    wid = sid * _NC + cid

    pltpu.sync_copy(
        z_hbm.at[pl.ds(sid * _SEG_PER_TILE, _SEG_PER_TILE), :],
        acc.at[pl.ds(sid * _SEG_PER_TILE, _SEG_PER_TILE), :],
    )
    plsc.subcore_barrier()

    base_w = pl.multiple_of(
        _G * (_BASE_GROUPS * wid + jnp.minimum(wid, _EXTRA)), _G
    )

    def process(base, ngroups, nrows):
        pltpu.sync_copy(x_hbm.at[pl.ds(base, nrows)], xv.at[pl.ds(0, nrows)])
        pltpu.sync_copy(y_hbm.at[pl.ds(base, nrows)], yv.at[pl.ds(0, nrows)])
        pltpu.sync_copy(p_hbm.at[pl.ds(base, nrows), :],
                        rows.at[pl.ds(0, nrows), :])
        for g in range(ngroups):
            for i in range(_G // 16):
                off = g * _G + i * 16
                seg = xv[pl.ds(off, 16)] * _Y + yv[pl.ds(off, 16)]
                idxv[g, pl.ds(i * 16, 16)] = seg
        for g in range(ngroups):
            pltpu.sync_copy(rows.at[pl.ds(g * _G, _G), :],
                            acc.at[idxv.at[g]], add=True)

    def chunk_body(j, carry):
        process(base_w + j * _CHUNK, _CG, _CHUNK)
        return carry

    lax.fori_loop(0, _NCHUNKS, chunk_body, 0)

    @pl.when(wid < _EXTRA)
    def _tail():
        process(base_w + _BASE_GROUPS * _G, 1, _G)

    plsc.subcore_barrier()
    pltpu.sync_copy(
        acc.at[pl.ds(sid * _SEG_PER_TILE, _SEG_PER_TILE), :],
        out_hbm.at[cid, pl.ds(sid * _SEG_PER_TILE, _SEG_PER_TILE), :],
    )


_sc_call = pl.kernel(
    _sc_body,
    out_type=jax.ShapeDtypeStruct((_NC, _NSEG, _C), jnp.float32),
    mesh=plsc.VectorSubcoreMesh(core_axis_name="c", subcore_axis_name="s"),
    scratch_types=[
        pltpu.VMEM((_CHUNK,), jnp.int32),
        pltpu.VMEM((_CHUNK,), jnp.int32),
        pltpu.VMEM((_CG, _G), jnp.int32),
        pltpu.VMEM((_CHUNK, _C), jnp.float32),
        pltpu.VMEM_SHARED((_NSEG, _C), jnp.float32),
    ],
)


def _merge_body(em_ref, p_ref, o_ref):
    o_ref[...] = em_ref[...] + p_ref[0] + p_ref[1]


_merge = pl.pallas_call(
    _merge_body,
    out_shape=jax.ShapeDtypeStruct((_NSEG, _C), jnp.float32),
)


def kernel(x_labels, y_labels, posterior_estimate, emission_numerator):
    zeros = jnp.zeros((_NSEG, _C), jnp.float32)
    partials = _sc_call(x_labels, y_labels, posterior_estimate, zeros)
    out = _merge(emission_numerator.reshape(_NSEG, _C), partials)
    return out.reshape(_K, _Y, _C)


# speedup vs baseline: 9.1134x; 1.5849x over previous
import jax
import jax.numpy as jnp
from jax import lax
from jax.experimental import pallas as pl
from jax.experimental.pallas import tpu as pltpu
from jax.experimental.pallas import tpu_sc as plsc

_K = 128
_Y = 16
_C = 128
_N = 320000
_NSEG = _K * _Y
_NC = 2
_NS = 16
_NW = _NC * _NS
_G = 128
_NGROUPS = _N // _G
_BASE_GROUPS = _NGROUPS // _NW
_EXTRA = _NGROUPS - _BASE_GROUPS * _NW
_CG = 3
_CHUNK = _CG * _G
_NCHUNKS = _BASE_GROUPS // _CG
_SEG_PER_TILE = _NSEG // _NS

assert _BASE_GROUPS % _CG == 0
assert _NGROUPS * _G == _N


def _sc_body(x_hbm, y_hbm, p_hbm, z_hbm, out_hbm,
             xv, yv, idxv, rows, acc, sem0, sem1):
    cid = lax.axis_index("c")
    sid = lax.axis_index("s")
    wid = sid * _NC + cid
    sems = (sem0, sem1)

    pltpu.sync_copy(
        z_hbm.at[pl.ds(sid * _SEG_PER_TILE, _SEG_PER_TILE), :],
        acc.at[pl.ds(sid * _SEG_PER_TILE, _SEG_PER_TILE), :],
    )
    plsc.subcore_barrier()

    base_w = pl.multiple_of(
        _G * (_BASE_GROUPS * wid + jnp.minimum(wid, _EXTRA)), _G
    )

    def start(c, b):
        base = base_w + c * _CHUNK
        pltpu.async_copy(x_hbm.at[pl.ds(base, _CHUNK)], xv.at[b], sems[b])
        pltpu.async_copy(y_hbm.at[pl.ds(base, _CHUNK)], yv.at[b], sems[b])
        pltpu.async_copy(p_hbm.at[pl.ds(base, _CHUNK), :], rows.at[b],
                         sems[b])

    def wait(b):
        pltpu.make_async_copy(x_hbm.at[pl.ds(0, _CHUNK)], xv.at[b],
                              sems[b]).wait()
        pltpu.make_async_copy(y_hbm.at[pl.ds(0, _CHUNK)], yv.at[b],
                              sems[b]).wait()
        pltpu.make_async_copy(p_hbm.at[pl.ds(0, _CHUNK), :], rows.at[b],
                              sems[b]).wait()

    def scatter(b):
        for g in range(_CG):
            for i in range(_G // 16):
                off = g * _G + i * 16
                seg = xv[b, pl.ds(off, 16)] * _Y + yv[b, pl.ds(off, 16)]
                idxv[g, pl.ds(i * 16, 16)] = seg
        for g in range(_CG):
            pltpu.sync_copy(rows.at[b, pl.ds(g * _G, _G), :],
                            acc.at[idxv.at[g]], add=True)

    start(0, 0)

    def pair_body(j, carry):
        for b in range(2):
            c = 2 * j + b

            @pl.when(c + 1 < _NCHUNKS)
            def _start_next():
                start(c + 1, 1 - b)

            wait(b)
            scatter(b)
        return carry

    lax.fori_loop(0, _NCHUNKS // 2, pair_body, 0)

    @pl.when(wid < _EXTRA)
    def _tail():
        base = base_w + _BASE_GROUPS * _G
        pltpu.sync_copy(x_hbm.at[pl.ds(base, _G)], xv.at[0, pl.ds(0, _G)])
        pltpu.sync_copy(y_hbm.at[pl.ds(base, _G)], yv.at[0, pl.ds(0, _G)])
        pltpu.sync_copy(p_hbm.at[pl.ds(base, _G), :],
                        rows.at[0, pl.ds(0, _G), :])
        for i in range(_G // 16):
            seg = xv[0, pl.ds(i * 16, 16)] * _Y + yv[0, pl.ds(i * 16, 16)]
            idxv[0, pl.ds(i * 16, 16)] = seg
        pltpu.sync_copy(rows.at[0, pl.ds(0, _G), :],
                        acc.at[idxv.at[0]], add=True)

    plsc.subcore_barrier()
    pltpu.sync_copy(
        acc.at[pl.ds(sid * _SEG_PER_TILE, _SEG_PER_TILE), :],
        out_hbm.at[cid, pl.ds(sid * _SEG_PER_TILE, _SEG_PER_TILE), :],
    )


_sc_call = pl.kernel(
    _sc_body,
    out_type=jax.ShapeDtypeStruct((_NC, _NSEG, _C), jnp.float32),
    mesh=plsc.VectorSubcoreMesh(core_axis_name="c", subcore_axis_name="s"),
    scratch_types=[
        pltpu.VMEM((2, _CHUNK), jnp.int32),
        pltpu.VMEM((2, _CHUNK), jnp.int32),
        pltpu.VMEM((_CG, _G), jnp.int32),
        pltpu.VMEM((2, _CHUNK, _C), jnp.float32),
        pltpu.VMEM_SHARED((_NSEG, _C), jnp.float32),
        pltpu.SemaphoreType.DMA,
        pltpu.SemaphoreType.DMA,
    ],
)


def _merge_body(em_ref, p_ref, o_ref):
    o_ref[...] = em_ref[...] + p_ref[0] + p_ref[1]


_merge = pl.pallas_call(
    _merge_body,
    out_shape=jax.ShapeDtypeStruct((_NSEG, _C), jnp.float32),
)


def kernel(x_labels, y_labels, posterior_estimate, emission_numerator):
    zeros = jnp.zeros((_NSEG, _C), jnp.float32)
    partials = _sc_call(x_labels, y_labels, posterior_estimate, zeros)
    out = _merge(emission_numerator.reshape(_NSEG, _C), partials)
    return out.reshape(_K, _Y, _C)
